# Initial kernel scaffold; baseline (speedup 1.0000x reference)
import jax
import jax.numpy as jnp
from jax.experimental import pallas as pl


def kernel(pillar_features, pillar_coords, batch_size):
    raise NotImplementedError("write your pallas kernel here")


# speedup vs baseline: 1.6375x; 1.6375x over previous
import functools

import jax
import jax.numpy as jnp
from jax import lax
from jax.experimental import pallas as pl
from jax.experimental.pallas import tpu as pltpu
from jax.experimental.pallas import tpu_sc as plsc

P = 50000
B = 2
C = 64
GX = 512
GY = 512
N_WORDS = B * C * GX * GY
XSTR = GX * GY
BSTR = C * GX * GY

NC, NS, L = 2, 16, 16
NWORK = NC * NS
PER_W = 1568
P_PAD = PER_W * NWORK
KC = 784
NCHUNK = PER_W // KC
GRP = KC // L


def _zero_canvas():
    def body(o_ref):
        o_ref[...] = jnp.zeros_like(o_ref)

    return pl.pallas_call(
        body,
        out_shape=jax.ShapeDtypeStruct((8192, 4096), jnp.float32),
        grid=(32,),
        out_specs=pl.BlockSpec((256, 4096), lambda i: (i, 0)),
    )()


_mesh = plsc.VectorSubcoreMesh(core_axis_name="c", subcore_axis_name="s")


@functools.partial(
    pl.kernel,
    out_type=(),
    mesh=_mesh,
    scratch_types=[
        pltpu.VMEM((PER_W,), jnp.int32),
        pltpu.VMEM((PER_W,), jnp.int32),
        pltpu.VMEM((PER_W,), jnp.int32),
        pltpu.VMEM((PER_W,), jnp.int32),
        pltpu.VMEM((KC * C,), jnp.float32),
        pltpu.VMEM((KC * C,), jnp.int32),
        pltpu.SemaphoreType.DMA,
    ],
)
def _sc_scatter(feat_hbm, b_hbm, x_hbm, y_hbm, out_ref,
                b_v, x_v, y_v, base_v, feat_v, idx_v, sem):
    wid = lax.axis_index("s") * NC + lax.axis_index("c")
    p0 = wid * PER_W
    pltpu.sync_copy(b_hbm.at[pl.ds(p0, PER_W)], b_v)
    pltpu.sync_copy(x_hbm.at[pl.ds(p0, PER_W)], x_v)
    pltpu.sync_copy(y_hbm.at[pl.ds(p0, PER_W)], y_v)

    def mkbase(g, carry):
        sl = pl.ds(g * L, L)
        base_v[sl] = b_v[sl] * BSTR + x_v[sl] * GY + y_v[sl]
        return carry

    lax.fori_loop(0, PER_W // L, mkbase, 0)

    cvec = [(lax.iota(jnp.int32, L) + 16 * k) * XSTR for k in range(C // L)]
    for ck in range(NCHUNK):
        c0 = ck * KC
        pltpu.sync_copy(feat_hbm.at[pl.ds((p0 + c0) * C, KC * C)], feat_v)

        def build(g, carry):
            bvec = base_v[pl.ds(c0 + g * L, L)]
            for j in range(L):
                s = bvec[j]
                row = (g * L + j) * C
                for k in range(C // L):
                    idx_v[pl.ds(row + k * L, L)] = cvec[k] + s
            return carry

        lax.fori_loop(0, GRP, build, 0)
        pltpu.async_copy(feat_v, out_ref.at[idx_v], sem).wait()


def kernel(pillar_features, pillar_coords, batch_size):
    del batch_size
    pad = P_PAD - P
    b = pillar_coords[:, 0]
    x = pillar_coords[:, 1]
    y = pillar_coords[:, 2]
    featp = jnp.concatenate(
        [pillar_features, jnp.broadcast_to(pillar_features[0], (pad, C))], 0
    ).reshape(P_PAD * C)
    bp = jnp.concatenate([b, jnp.broadcast_to(b[0], (pad,))])
    xp = jnp.concatenate([x, jnp.broadcast_to(x[0], (pad,))])
    yp = jnp.concatenate([y, jnp.broadcast_to(y[0], (pad,))])
    z = _zero_canvas().reshape(N_WORDS)
    acc = jax.new_ref(z)
    _sc_scatter(featp, bp, xp, yp, acc)
    return jax.freeze(acc).reshape(B, C, GX, GY)


# speedup vs baseline: 10.3433x; 6.3166x over previous
import functools

import jax
import jax.numpy as jnp
from jax import lax
from jax.experimental import pallas as pl
from jax.experimental.pallas import tpu as pltpu
from jax.experimental.pallas import tpu_sc as plsc

P = 50000
B = 2
C = 64
GX = 512
GY = 512
NCELL = B * GX * GY

NC, NS, L = 2, 16, 16
NWORK = NC * NS
PER_W = 1568
P_PAD = PER_W * NWORK
GRP = PER_W // L

XB = 8


def _zero_mask():
    def body(o_ref):
        o_ref[...] = jnp.zeros_like(o_ref)

    return pl.pallas_call(
        body,
        out_shape=jax.ShapeDtypeStruct((B * GX, GY), jnp.int32),
        grid=(2,),
        out_specs=pl.BlockSpec((B * GX // 2, GY), lambda i: (i, 0)),
    )()


_mesh = plsc.VectorSubcoreMesh(core_axis_name="c", subcore_axis_name="s")


@functools.partial(
    pl.kernel,
    out_type=jax.ShapeDtypeStruct((NCELL, C), jnp.float32),
    mesh=_mesh,
    compiler_params=pltpu.CompilerParams(use_tc_tiling_on_sc=False),
    scratch_types=[
        pltpu.VMEM((PER_W,), jnp.int32),
        pltpu.VMEM((PER_W,), jnp.int32),
        pltpu.VMEM((PER_W,), jnp.int32),
        pltpu.VMEM((PER_W,), jnp.int32),
        pltpu.VMEM((PER_W,), jnp.int32),
        pltpu.VMEM((PER_W, C), jnp.float32),
        pltpu.SemaphoreType.DMA,
        pltpu.SemaphoreType.DMA,
    ],
)
def _sc_scatter(feat_hbm, b_hbm, x_hbm, y_hbm, mask_ref, t_ref,
                b_v, x_v, y_v, q_v, ones_v, feat_v, sem_t, sem_m):
    wid = lax.axis_index("s") * NC + lax.axis_index("c")
    p0 = wid * PER_W
    cp_feat = pltpu.async_copy(feat_hbm.at[pl.ds(p0, PER_W)], feat_v, sem_t)
    pltpu.sync_copy(b_hbm.at[pl.ds(p0, PER_W)], b_v)
    pltpu.sync_copy(x_hbm.at[pl.ds(p0, PER_W)], x_v)
    pltpu.sync_copy(y_hbm.at[pl.ds(p0, PER_W)], y_v)

    def build(g, carry):
        sl = pl.ds(g * L, L)
        q_v[sl] = b_v[sl] * (GX * GY) + x_v[sl] * GY + y_v[sl]
        ones_v[sl] = jnp.ones((L,), jnp.int32)
        return carry

    lax.fori_loop(0, GRP, build, 0)
    cp_feat.wait()
    cp_mask = pltpu.async_copy(ones_v, mask_ref.at[q_v], sem_m)
    pltpu.async_copy(feat_v, t_ref.at[q_v], sem_t).wait()
    cp_mask.wait()


def _transpose_masked(mask2d, t):
    def body(m_ref, t_ref, o_ref):
        tt = jnp.transpose(t_ref[...], (1, 0))
        m = m_ref[...].reshape(1, XB, GY)
        o_ref[...] = jnp.where(m != 0, tt.reshape(C, XB, GY), 0.0)[None]

    return pl.pallas_call(
        body,
        grid=(B * GX // XB,),
        in_specs=[
            pl.BlockSpec((XB, GY), lambda g: (g, 0)),
            pl.BlockSpec((XB * GY, C), lambda g: (g, 0)),
        ],
        out_specs=pl.BlockSpec(
            (1, C, XB, GY),
            lambda g: (g // (GX // XB), 0, g % (GX // XB), 0),
        ),
        out_shape=jax.ShapeDtypeStruct((B, C, GX, GY), jnp.float32),
    )(mask2d, t)


def kernel(pillar_features, pillar_coords, batch_size):
    del batch_size
    pad = P_PAD - P
    b = pillar_coords[:, 0]
    x = pillar_coords[:, 1]
    y = pillar_coords[:, 2]
    featp = jnp.concatenate(
        [pillar_features, jnp.broadcast_to(pillar_features[0], (pad, C))], 0
    )
    bp = jnp.concatenate([b, jnp.broadcast_to(b[0], (pad,))])
    xp = jnp.concatenate([x, jnp.broadcast_to(x[0], (pad,))])
    yp = jnp.concatenate([y, jnp.broadcast_to(y[0], (pad,))])
    mask_ref = jax.new_ref(_zero_mask().reshape(NCELL))
    t = _sc_scatter(featp, bp, xp, yp, mask_ref)
    mask2d = jax.freeze(mask_ref).reshape(B * GX, GY)
    return _transpose_masked(mask2d, t)


# speedup vs baseline: 10.4374x; 1.0091x over previous
import functools

import jax
import jax.numpy as jnp
from jax import lax
from jax.experimental import pallas as pl
from jax.experimental.pallas import tpu as pltpu
from jax.experimental.pallas import tpu_sc as plsc

P = 50000
B = 2
C = 64
GX = 512
GY = 512
NCELL = B * GX * GY

NC, NS, L = 2, 16, 16
NWORK = NC * NS
PER_W = 1568
P_PAD = PER_W * NWORK
GRP = PER_W // L

XB = 8


def _zero_mask():
    def body(o_ref):
        o_ref[...] = jnp.zeros_like(o_ref)

    return pl.pallas_call(
        body,
        out_shape=jax.ShapeDtypeStruct((B * GX, GY), jnp.int32),
        grid=(2,),
        out_specs=pl.BlockSpec((B * GX // 2, GY), lambda i: (i, 0)),
    )()


_mesh = plsc.VectorSubcoreMesh(core_axis_name="c", subcore_axis_name="s")


@functools.partial(
    pl.kernel,
    out_type=jax.ShapeDtypeStruct((NCELL, C), jnp.float32),
    mesh=_mesh,
    compiler_params=pltpu.CompilerParams(use_tc_tiling_on_sc=False),
    scratch_types=[
        pltpu.VMEM((PER_W,), jnp.int32),
        pltpu.VMEM((PER_W,), jnp.int32),
        pltpu.VMEM((PER_W,), jnp.int32),
        pltpu.VMEM((PER_W,), jnp.int32),
        pltpu.VMEM((PER_W,), jnp.int32),
        pltpu.VMEM((PER_W, C), jnp.float32),
        pltpu.SemaphoreType.DMA,
        pltpu.SemaphoreType.DMA,
    ],
)
def _sc_scatter(feat_hbm, b_hbm, x_hbm, y_hbm, mask_ref, t_ref,
                b_v, x_v, y_v, q_v, ones_v, feat_v, sem_t, sem_m):
    wid = lax.axis_index("s") * NC + lax.axis_index("c")
    p0 = wid * PER_W
    cp_feat = pltpu.async_copy(feat_hbm.at[pl.ds(p0, PER_W)], feat_v, sem_t)
    pltpu.sync_copy(b_hbm.at[pl.ds(p0, PER_W)], b_v)
    pltpu.sync_copy(x_hbm.at[pl.ds(p0, PER_W)], x_v)
    pltpu.sync_copy(y_hbm.at[pl.ds(p0, PER_W)], y_v)

    def build(g, carry):
        sl = pl.ds(g * L, L)
        q_v[sl] = b_v[sl] * (GX * GY) + x_v[sl] * GY + y_v[sl]
        ones_v[sl] = jnp.ones((L,), jnp.int32)
        return carry

    lax.fori_loop(0, GRP, build, 0)
    cp_feat.wait()
    cp_mask = pltpu.async_copy(ones_v, mask_ref.at[q_v], sem_m)
    pltpu.async_copy(feat_v, t_ref.at[q_v], sem_t).wait()
    cp_mask.wait()


def _transpose_masked(mask2d, t):
    def body(m_ref, t_ref, o_ref):
        tt = lax.dot_general(
            jnp.eye(C, dtype=jnp.float32), t_ref[...],
            (((1,), (1,)), ((), ())),
            preferred_element_type=jnp.float32,
        )
        m = m_ref[...].reshape(1, XB, GY)
        o_ref[...] = jnp.where(m != 0, tt.reshape(C, XB, GY), 0.0)[None]

    return pl.pallas_call(
        body,
        grid=(B * GX // XB,),
        in_specs=[
            pl.BlockSpec((XB, GY), lambda g: (g, 0)),
            pl.BlockSpec((XB * GY, C), lambda g: (g, 0)),
        ],
        out_specs=pl.BlockSpec(
            (1, C, XB, GY),
            lambda g: (g // (GX // XB), 0, g % (GX // XB), 0),
        ),
        out_shape=jax.ShapeDtypeStruct((B, C, GX, GY), jnp.float32),
    )(mask2d, t)


def kernel(pillar_features, pillar_coords, batch_size):
    del batch_size
    pad = P_PAD - P
    b = pillar_coords[:, 0]
    x = pillar_coords[:, 1]
    y = pillar_coords[:, 2]
    featp = jnp.concatenate(
        [pillar_features, jnp.broadcast_to(pillar_features[0], (pad, C))], 0
    )
    bp = jnp.concatenate([b, jnp.broadcast_to(b[0], (pad,))])
    xp = jnp.concatenate([x, jnp.broadcast_to(x[0], (pad,))])
    yp = jnp.concatenate([y, jnp.broadcast_to(y[0], (pad,))])
    mask_ref = jax.new_ref(_zero_mask().reshape(NCELL))
    t = _sc_scatter(featp, bp, xp, yp, mask_ref)
    mask2d = jax.freeze(mask_ref).reshape(B * GX, GY)
    return _transpose_masked(mask2d, t)


# speedup vs baseline: 11.2007x; 1.0731x over previous
import functools

import jax
import jax.numpy as jnp
from jax import lax
from jax.experimental import pallas as pl
from jax.experimental.pallas import tpu as pltpu
from jax.experimental.pallas import tpu_sc as plsc

P = 50000
B = 2
C = 64
GX = 512
GY = 512
NCELL = B * GX * GY

NC, NS, L = 2, 16, 16
NWORK = NC * NS
PER_W = 1568
P_PAD = PER_W * NWORK
GRP = PER_W // L

XB = 16


def _zero_mask():
    def body(o_ref):
        o_ref[...] = jnp.zeros_like(o_ref)

    return pl.pallas_call(
        body,
        out_shape=jax.ShapeDtypeStruct((B * GX, GY), jnp.int32),
        grid=(2,),
        out_specs=pl.BlockSpec((B * GX // 2, GY), lambda i: (i, 0)),
    )()


_mesh = plsc.VectorSubcoreMesh(core_axis_name="c", subcore_axis_name="s")


@functools.partial(
    pl.kernel,
    out_type=jax.ShapeDtypeStruct((NCELL, C), jnp.float32),
    mesh=_mesh,
    compiler_params=pltpu.CompilerParams(use_tc_tiling_on_sc=False),
    scratch_types=[
        pltpu.VMEM((PER_W,), jnp.int32),
        pltpu.VMEM((PER_W,), jnp.int32),
        pltpu.VMEM((PER_W,), jnp.int32),
        pltpu.VMEM((PER_W,), jnp.int32),
        pltpu.VMEM((PER_W,), jnp.int32),
        pltpu.VMEM((PER_W, C), jnp.float32),
        pltpu.SemaphoreType.DMA,
        pltpu.SemaphoreType.DMA,
    ],
)
def _sc_scatter(feat_hbm, b_hbm, x_hbm, y_hbm, mask_ref, t_ref,
                b_v, x_v, y_v, q_v, ones_v, feat_v, sem_t, sem_m):
    wid = lax.axis_index("s") * NC + lax.axis_index("c")
    p0 = wid * PER_W
    cp_feat = pltpu.async_copy(feat_hbm.at[pl.ds(p0, PER_W)], feat_v, sem_t)
    pltpu.sync_copy(b_hbm.at[pl.ds(p0, PER_W)], b_v)
    pltpu.sync_copy(x_hbm.at[pl.ds(p0, PER_W)], x_v)
    pltpu.sync_copy(y_hbm.at[pl.ds(p0, PER_W)], y_v)

    def build(g, carry):
        sl = pl.ds(g * L, L)
        q_v[sl] = b_v[sl] * (GX * GY) + x_v[sl] * GY + y_v[sl]
        ones_v[sl] = jnp.ones((L,), jnp.int32)
        return carry

    lax.fori_loop(0, GRP, build, 0)
    cp_feat.wait()
    cp_mask = pltpu.async_copy(ones_v, mask_ref.at[q_v], sem_m)
    pltpu.async_copy(feat_v, t_ref.at[q_v], sem_t).wait()
    cp_mask.wait()


def _transpose_masked(mask2d, t):
    def body(m_ref, t_ref, o_ref):
        tt = jnp.transpose(t_ref[...], (1, 0))
        m = m_ref[...].reshape(1, XB, GY)
        o_ref[...] = jnp.where(m != 0, tt.reshape(C, XB, GY), 0.0)[None]

    return pl.pallas_call(
        body,
        grid=(B * GX // XB,),
        in_specs=[
            pl.BlockSpec((XB, GY), lambda g: (g, 0)),
            pl.BlockSpec((XB * GY, C), lambda g: (g, 0)),
        ],
        out_specs=pl.BlockSpec(
            (1, C, XB, GY),
            lambda g: (g // (GX // XB), 0, g % (GX // XB), 0),
        ),
        out_shape=jax.ShapeDtypeStruct((B, C, GX, GY), jnp.float32),
    )(mask2d, t)


def kernel(pillar_features, pillar_coords, batch_size):
    del batch_size
    pad = P_PAD - P
    b = pillar_coords[:, 0]
    x = pillar_coords[:, 1]
    y = pillar_coords[:, 2]
    featp = jnp.concatenate(
        [pillar_features, jnp.broadcast_to(pillar_features[0], (pad, C))], 0
    )
    bp = jnp.concatenate([b, jnp.broadcast_to(b[0], (pad,))])
    xp = jnp.concatenate([x, jnp.broadcast_to(x[0], (pad,))])
    yp = jnp.concatenate([y, jnp.broadcast_to(y[0], (pad,))])
    mask_ref = jax.new_ref(_zero_mask().reshape(NCELL))
    t = _sc_scatter(featp, bp, xp, yp, mask_ref)
    mask2d = jax.freeze(mask_ref).reshape(B * GX, GY)
    return _transpose_masked(mask2d, t)


# speedup vs baseline: 11.3314x; 1.0117x over previous
import functools

import jax
import jax.numpy as jnp
from jax import lax
from jax.experimental import pallas as pl
from jax.experimental.pallas import tpu as pltpu
from jax.experimental.pallas import tpu_sc as plsc

P = 50000
B = 2
C = 64
GX = 512
GY = 512
NCELL = B * GX * GY

NC, NS, L = 2, 16, 16
NWORK = NC * NS
PER_W = 1568
P_PAD = PER_W * NWORK
GRP = PER_W // L

XB = 32


def _zero_mask():
    def body(o_ref):
        o_ref[...] = jnp.zeros_like(o_ref)

    return pl.pallas_call(
        body,
        out_shape=jax.ShapeDtypeStruct((B * GX, GY), jnp.int32),
        grid=(2,),
        out_specs=pl.BlockSpec((B * GX // 2, GY), lambda i: (i, 0)),
    )()


_mesh = plsc.VectorSubcoreMesh(core_axis_name="c", subcore_axis_name="s")


@functools.partial(
    pl.kernel,
    out_type=jax.ShapeDtypeStruct((NCELL, C), jnp.float32),
    mesh=_mesh,
    compiler_params=pltpu.CompilerParams(use_tc_tiling_on_sc=False),
    scratch_types=[
        pltpu.VMEM((PER_W,), jnp.int32),
        pltpu.VMEM((PER_W,), jnp.int32),
        pltpu.VMEM((PER_W,), jnp.int32),
        pltpu.VMEM((PER_W,), jnp.int32),
        pltpu.VMEM((PER_W,), jnp.int32),
        pltpu.VMEM((PER_W, C), jnp.float32),
        pltpu.SemaphoreType.DMA,
        pltpu.SemaphoreType.DMA,
    ],
)
def _sc_scatter(feat_hbm, b_hbm, x_hbm, y_hbm, mask_ref, t_ref,
                b_v, x_v, y_v, q_v, ones_v, feat_v, sem_t, sem_m):
    wid = lax.axis_index("s") * NC + lax.axis_index("c")
    p0 = wid * PER_W
    cp_feat = pltpu.async_copy(feat_hbm.at[pl.ds(p0, PER_W)], feat_v, sem_t)
    pltpu.sync_copy(b_hbm.at[pl.ds(p0, PER_W)], b_v)
    pltpu.sync_copy(x_hbm.at[pl.ds(p0, PER_W)], x_v)
    pltpu.sync_copy(y_hbm.at[pl.ds(p0, PER_W)], y_v)

    def build(g, carry):
        sl = pl.ds(g * L, L)
        q_v[sl] = b_v[sl] * (GX * GY) + x_v[sl] * GY + y_v[sl]
        ones_v[sl] = jnp.ones((L,), jnp.int32)
        return carry

    lax.fori_loop(0, GRP, build, 0)
    cp_feat.wait()
    cp_mask = pltpu.async_copy(ones_v, mask_ref.at[q_v], sem_m)
    pltpu.async_copy(feat_v, t_ref.at[q_v], sem_t).wait()
    cp_mask.wait()


def _transpose_masked(mask2d, t):
    def body(m_ref, t_ref, o_ref):
        tt = jnp.transpose(t_ref[...], (1, 0))
        m = m_ref[...].reshape(1, XB, GY)
        o_ref[...] = jnp.where(m != 0, tt.reshape(C, XB, GY), 0.0)[None]

    return pl.pallas_call(
        body,
        grid=(B * GX // XB,),
        in_specs=[
            pl.BlockSpec((XB, GY), lambda g: (g, 0)),
            pl.BlockSpec((XB * GY, C), lambda g: (g, 0)),
        ],
        out_specs=pl.BlockSpec(
            (1, C, XB, GY),
            lambda g: (g // (GX // XB), 0, g % (GX // XB), 0),
        ),
        out_shape=jax.ShapeDtypeStruct((B, C, GX, GY), jnp.float32),
    )(mask2d, t)


def kernel(pillar_features, pillar_coords, batch_size):
    del batch_size
    pad = P_PAD - P
    b = pillar_coords[:, 0]
    x = pillar_coords[:, 1]
    y = pillar_coords[:, 2]
    featp = jnp.concatenate(
        [pillar_features, jnp.broadcast_to(pillar_features[0], (pad, C))], 0
    )
    bp = jnp.concatenate([b, jnp.broadcast_to(b[0], (pad,))])
    xp = jnp.concatenate([x, jnp.broadcast_to(x[0], (pad,))])
    yp = jnp.concatenate([y, jnp.broadcast_to(y[0], (pad,))])
    mask_ref = jax.new_ref(_zero_mask().reshape(NCELL))
    t = _sc_scatter(featp, bp, xp, yp, mask_ref)
    mask2d = jax.freeze(mask_ref).reshape(B * GX, GY)
    return _transpose_masked(mask2d, t)
